# Initial kernel scaffold; baseline (speedup 1.0000x reference)
import jax
import jax.numpy as jnp
from jax.experimental import pallas as pl


def kernel(x_s, x_t, params, edge_index_s, batch_s, edge_index_t, batch_t):
    raise NotImplementedError("write your pallas kernel here")


# speedup vs baseline: 13.5242x; 13.5242x over previous
import functools

import jax
import jax.numpy as jnp
from jax import lax
from jax.experimental import pallas as pl
from jax.experimental.pallas import tpu as pltpu
from jax.experimental.pallas import tpu_sc as plsc

N = 10000
E = 320000
D_IN = 128
D = 64
NC, NS = 2, 16
CHUNK = 128
EPW = 158
EPAD = NS * EPW * CHUNK
NRP = 10112
NPS = NRP // NS
NPAD = NRP + 16

def _spmm_body(a_hbm, srcg_hbm, dst_hbm, zeros_hbm, out_hbm,
               src_v, dst_v, buf0, buf1, s_sh, sem0, sem1):
    c = lax.axis_index("c")
    s = lax.axis_index("s")
    r0 = s * NPS
    pltpu.sync_copy(zeros_hbm.at[pl.ds(r0, NPS)], s_sh.at[pl.ds(r0, NPS)])
    pltpu.sync_copy(srcg_hbm.at[c, s], src_v)
    pltpu.sync_copy(dst_hbm.at[c, s], dst_v)
    plsc.subcore_barrier()
    bufs = (buf0, buf1)
    sems = (sem0, sem1)
    for b in range(2):
        pltpu.async_copy(a_hbm.at[src_v.at[b]], bufs[b], sems[b])

    def grp(g, carry):
        for b in range(2):
            j = 2 * g + b
            pltpu.make_async_copy(a_hbm.at[src_v.at[j]], bufs[b], sems[b]).wait()
            pltpu.sync_copy(bufs[b], s_sh.at[dst_v.at[j]], add=True)
            pltpu.async_copy(a_hbm.at[src_v.at[j + 2]], bufs[b], sems[b])
        return carry

    lax.fori_loop(0, EPW // 2 - 1, grp, 0)
    for b in range(2):
        j = EPW - 2 + b
        pltpu.make_async_copy(a_hbm.at[src_v.at[j]], bufs[b], sems[b]).wait()
        pltpu.sync_copy(bufs[b], s_sh.at[dst_v.at[j]], add=True)
    plsc.subcore_barrier()
    pltpu.sync_copy(s_sh.at[pl.ds(r0, NPS)], out_hbm.at[c, pl.ds(r0, NPS)])


def _degk_body(dst_hbm, zeros_hbm, out_hbm, dst_v, ones_v, d_sh):
    c = lax.axis_index("c")
    s = lax.axis_index("s")
    r0 = s * NPS
    pltpu.sync_copy(zeros_hbm.at[pl.ds(r0, NPS)], d_sh.at[pl.ds(r0, NPS)])
    pltpu.sync_copy(dst_hbm.at[c, s], dst_v)

    def fill(i, carry):
        ones_v[i, :] = jnp.ones((16,), jnp.float32)
        return carry

    lax.fori_loop(0, CHUNK, fill, 0)
    plsc.subcore_barrier()

    def body(j, carry):
        pltpu.sync_copy(ones_v, d_sh.at[dst_v.at[j]], add=True)
        return carry

    lax.fori_loop(0, EPW, body, 0)
    plsc.subcore_barrier()
    pltpu.sync_copy(d_sh.at[pl.ds(r0, NPS)], out_hbm.at[c, pl.ds(r0, NPS)])


_SC_KERNELS = None


def _get_sc_kernels():
    global _SC_KERNELS
    if _SC_KERNELS is None:
        mesh = plsc.VectorSubcoreMesh(core_axis_name="c", subcore_axis_name="s",
                                      num_cores=NC, num_subcores=NS)
        spmm = pl.kernel(
            _spmm_body,
            out_type=jax.ShapeDtypeStruct((NC, NRP, D), jnp.float32),
            mesh=mesh,
            compiler_params=pltpu.CompilerParams(use_tc_tiling_on_sc=False),
            scratch_types=[
                pltpu.VMEM((EPW, CHUNK), jnp.int32),
                pltpu.VMEM((EPW, CHUNK), jnp.int32),
                pltpu.VMEM((CHUNK, D), jnp.float32),
                pltpu.VMEM((CHUNK, D), jnp.float32),
                pltpu.VMEM_SHARED((NPAD, D), jnp.float32),
                pltpu.SemaphoreType.DMA,
                pltpu.SemaphoreType.DMA,
            ],
        )
        degk = pl.kernel(
            _degk_body,
            out_type=jax.ShapeDtypeStruct((NC, NRP, 16), jnp.float32),
            mesh=mesh,
            compiler_params=pltpu.CompilerParams(use_tc_tiling_on_sc=False),
            scratch_types=[
                pltpu.VMEM((EPW, CHUNK), jnp.int32),
                pltpu.VMEM((CHUNK, 16), jnp.float32),
                pltpu.VMEM_SHARED((NPAD, 16), jnp.float32),
            ],
        )
        _SC_KERNELS = (spmm, degk)
    return _SC_KERNELS


def _spmm(a_flat, srcg, dstp, z64):
    return _get_sc_kernels()[0](a_flat, srcg, dstp, z64)


def _degk(dstp, z16):
    return _get_sc_kernels()[1](dstp, z16)


def _pre_body(x_ref, deg_ref, w1, b1, w2, b2, w3, b3, a_ref, d_ref):
    h = x_ref[0]
    deg = deg_ref[0][:N, 0:1]
    a_ref[0] = jnp.dot(h, w1[...], preferred_element_type=jnp.float32) + b1[...]
    d_ref[0] = (jnp.dot(h, w3[...], preferred_element_type=jnp.float32) + b3[...]
                - deg * (jnp.dot(h, w2[...], preferred_element_type=jnp.float32)
                         + b2[...]))


def _bn_relu(h2, gam, bet):
    mean = jnp.mean(h2, axis=0, keepdims=True)
    var = jnp.mean((h2 - mean) ** 2, axis=0, keepdims=True)
    return jnp.maximum((h2 - mean) * (1.0 / jnp.sqrt(var + 1e-5)) * gam + bet,
                       0.0)


def _mid_body(s_ref, dp_ref, gam, bet, deg_ref, w1, b1, w2, b2, w3, b3,
              a_ref, d_ref):
    h = _bn_relu(s_ref[0][:N] + dp_ref[0], gam[...], bet[...])
    deg = deg_ref[0][:N, 0:1]
    a_ref[0] = jnp.dot(h, w1[...], preferred_element_type=jnp.float32) + b1[...]
    d_ref[0] = (jnp.dot(h, w3[...], preferred_element_type=jnp.float32) + b3[...]
                - deg * (jnp.dot(h, w2[...], preferred_element_type=jnp.float32)
                         + b2[...]))


def _final_body(s_ref, dp_ref, gam, bet, batch_ref,
                gw1, gb1, gw2, gb2, ow0, ob0, ow1, ob1, ow2, ob2, ow3, ob3,
                out_ref):
    embs = []
    for t in range(2):
        h = _bn_relu(s_ref[t][:N] + dp_ref[t], gam[...], bet[...])
        g1 = jnp.maximum(
            jnp.dot(h, gw1[...], preferred_element_type=jnp.float32) + gb1[...],
            0.0)
        gate = jnp.maximum(
            jnp.dot(g1, gw2[...], preferred_element_type=jnp.float32) + gb2[...],
            0.0)
        b = batch_ref[t]
        gids = lax.broadcasted_iota(jnp.int32, (N, 64), 1)
        m = b[:, None] == gids
        mf = m.astype(jnp.float32)
        scores = jnp.where(m, gate, -jnp.inf)
        gmax = jnp.max(scores, axis=0)
        gmax = jnp.where(jnp.isfinite(gmax), gmax, 0.0)
        gmax_n = jnp.dot(mf, gmax[:, None], preferred_element_type=jnp.float32)
        e = jnp.exp(gate - gmax_n)
        esum = lax.dot_general(mf, e, (((0,), (0,)), ((), ())),
                               preferred_element_type=jnp.float32)
        esum_n = jnp.dot(mf, esum, preferred_element_type=jnp.float32)
        attn = e / (esum_n + 1e-16)
        emb = lax.dot_general(mf, attn * h, (((0,), (0,)), ((), ())),
                              preferred_element_type=jnp.float32)
        embs.append(emb)
    h = jnp.abs(embs[0] - embs[1])
    h = jnp.maximum(jnp.dot(h, ow0[...], preferred_element_type=jnp.float32)
                    + ob0[...], 0.0)
    h = jnp.maximum(jnp.dot(h, ow1[...], preferred_element_type=jnp.float32)
                    + ob1[...], 0.0)
    h = jnp.maximum(jnp.dot(h, ow2[...], preferred_element_type=jnp.float32)
                    + ob2[...], 0.0)
    out_ref[...] = (jnp.dot(h, ow3[...], preferred_element_type=jnp.float32)
                    + ob3[...])


def _tower_spec(dim):
    return pl.BlockSpec((1, N, dim), lambda t: (t, 0, 0))


def _ptower_spec(dim):
    return pl.BlockSpec((1, NRP, dim), lambda t: (t, 0, 0))


def _full_spec(shape):
    nd = len(shape)
    return pl.BlockSpec(shape, lambda t: (0,) * nd)


def _pre_call(x, deg, w1, b1, w2, b2, w3, b3):
    return pl.pallas_call(
        _pre_body,
        grid=(2,),
        in_specs=[_tower_spec(D_IN), _ptower_spec(16),
                  _full_spec((D_IN, D)), _full_spec((1, D)),
                  _full_spec((D_IN, D)), _full_spec((1, D)),
                  _full_spec((D_IN, D)), _full_spec((1, D))],
        out_specs=[_tower_spec(D), _tower_spec(D)],
        out_shape=[jax.ShapeDtypeStruct((2, N, D), jnp.float32),
                   jax.ShapeDtypeStruct((2, N, D), jnp.float32)],
        compiler_params=pltpu.CompilerParams(
            dimension_semantics=("arbitrary",)),
    )(x, deg, w1, b1, w2, b2, w3, b3)


def _mid_call(s, d, gam, bet, deg, w1, b1, w2, b2, w3, b3):
    return pl.pallas_call(
        _mid_body,
        grid=(2,),
        in_specs=[_ptower_spec(D), _tower_spec(D),
                  _full_spec((1, D)), _full_spec((1, D)), _ptower_spec(16),
                  _full_spec((D, D)), _full_spec((1, D)),
                  _full_spec((D, D)), _full_spec((1, D)),
                  _full_spec((D, D)), _full_spec((1, D))],
        out_specs=[_tower_spec(D), _tower_spec(D)],
        out_shape=[jax.ShapeDtypeStruct((2, N, D), jnp.float32),
                   jax.ShapeDtypeStruct((2, N, D), jnp.float32)],
        compiler_params=pltpu.CompilerParams(
            dimension_semantics=("arbitrary",)),
    )(s, d, gam, bet, deg, w1, b1, w2, b2, w3, b3)


def _final_call(s, d, gam, bet, batch2, gw1, gb1, gw2, gb2, *outp):
    return pl.pallas_call(
        _final_body,
        out_shape=jax.ShapeDtypeStruct((64, 1), jnp.float32),
    )(s, d, gam, bet, batch2, gw1, gb1, gw2, gb2, *outp)


def kernel(x_s, x_t, params, edge_index_s, batch_s, edge_index_t, batch_t):
    i32 = jnp.int32
    f32 = jnp.float32
    srcs, dsts = [], []
    for t, ei in enumerate((edge_index_s, edge_index_t)):
        pad = EPAD - E
        src = jnp.concatenate([ei[0] + t * N, jnp.full((pad,), t * N, i32)])
        dst = jnp.concatenate([ei[1], jnp.full((pad,), NRP, i32)])
        srcs.append(src.reshape(NS, EPW, CHUNK))
        dsts.append(dst.reshape(NS, EPW, CHUNK))
    srcg = jnp.stack(srcs)
    dstp = jnp.stack(dsts)
    z16 = jnp.zeros((NRP, 16), f32)
    z64 = jnp.zeros((NRP, D), f32)
    deg = _degk(dstp, z16)
    x = jnp.stack([x_s, x_t])
    batch2 = jnp.stack([batch_s, batch_t])

    def wb(l):
        return (params["conv%d_W1" % l], params["conv%d_b1" % l][None, :],
                params["conv%d_W2" % l], params["conv%d_b2" % l][None, :],
                params["conv%d_W3" % l], params["conv%d_b3" % l][None, :])

    a, d = _pre_call(x, deg, *wb(0))
    for l in range(5):
        s = _spmm(a.reshape(NC * N, D), srcg, dstp, z64)
        a, d = _mid_call(s, d, params["conv%d_gamma" % l][None, :],
                         params["conv%d_beta" % l][None, :], deg, *wb(l + 1))
    s = _spmm(a.reshape(NC * N, D), srcg, dstp, z64)
    return _final_call(
        s, d, params["conv5_gamma"][None, :], params["conv5_beta"][None, :],
        batch2,
        params["gate_W1"], params["gate_b1"][None, :],
        params["gate_W2"], params["gate_b2"][None, :],
        params["out_W0"], params["out_b0"][None, :],
        params["out_W1"], params["out_b1"][None, :],
        params["out_W2"], params["out_b2"][None, :],
        params["out_W3"], params["out_b3"][None, :])
